# Initial kernel scaffold; baseline (speedup 1.0000x reference)
import jax
import jax.numpy as jnp
from jax.experimental import pallas as pl


def kernel(in_tensor, conditioning_code, tables):
    raise NotImplementedError("write your pallas kernel here")


# speedup vs baseline: 2.0697x; 2.0697x over previous
import functools
import numpy as np
import jax
import jax.numpy as jnp
from jax import lax
from jax.experimental import pallas as pl
from jax.experimental.pallas import tpu as pltpu
from jax.experimental.pallas import tpu_sc as plsc

_N_LEVELS = 16
_F = 2
_T = 2 ** 19
_BASE_RES = 16
_SCALE = 1.4472692012786865
_N_HASH = 4
_MASK = _T - 1
_P1 = -1640531535
_P2 = 805459861

_RES = [int(np.floor(_BASE_RES * _SCALE ** l)) for l in range(_N_LEVELS)]
_N_DENSE = sum(1 for r in _RES if (r + 1) ** 3 <= _T)

_B = 131072
_NC, _NS = 2, 16
_NW = _NC * _NS
_BW = _B // _NW
_CH = 512
_NSUB = _BW // _CH
_NG = _CH // 16
_NIDX = _CH * 8
_IDX_ROWS = _NIDX // 128


def _sc_body(tbl_hbm, x0_hbm, x1_hbm, x2_hbm, k0_hbm, k1_hbm, k2_hbm, k3_hbm,
             resf_hbm, stridei_hbm, out_hbm,
             idx_buf, rows_v, wc_v, x0_v, x1_v, x2_v,
             c0_v, c1_v, c2_v, c3_v, out_t, lvl_stage, resf_v, stridei_v, sem):
    wid = lax.axis_index("s") * _NC + lax.axis_index("c")
    iota = lax.iota(jnp.int32, 16)

    pltpu.sync_copy(resf_hbm, lvl_stage)
    rv = lvl_stage[pl.ds(0, 16)]
    pltpu.sync_copy(stridei_hbm, lvl_stage)
    sv = lvl_stage[pl.ds(0, 16)]
    for l in range(_N_LEVELS):
        resf_v[l] = rv[l]
        stridei_v[l] = sv[l].astype(jnp.int32)

    def pass1_group(g, l, lbase, resf, stride):
        x0 = x0_v[pl.ds(g * 16, 16)]
        x1 = x1_v[pl.ds(g * 16, 16)]
        x2 = x2_v[pl.ds(g * 16, 16)]
        p0 = (x0 * resf).astype(jnp.int32)
        p1 = (x1 * resf).astype(jnp.int32)
        p2 = (x2 * resf).astype(jnp.int32)
        w0 = x0 * resf - p0.astype(jnp.float32)
        w1 = x1 * resf - p1.astype(jnp.float32)
        w2 = x2 * resf - p2.astype(jnp.float32)
        m0 = 1.0 - w0
        m1 = 1.0 - w1
        m2 = 1.0 - w2
        if stride is None:
            a0 = p0
            b0 = p0 + 1
            a1 = p1 * _P1
            b1 = a1 + _P1
            a2 = p2 * _P2
            b2 = a2 + _P2
        else:
            s2 = stride * stride
            dbase = p0 + p1 * stride + p2 * s2 + lbase
        col = (g & 7) * 16
        for c in range(8):
            o0, o1, o2 = c & 1, (c >> 1) & 1, (c >> 2) & 1
            if stride is None:
                h = (b0 if o0 else a0) ^ (b1 if o1 else a1) ^ (b2 if o2 else a2)
                idx = (h & _MASK) + lbase
            else:
                idx = dbase + (o0 + stride * o1 + s2 * o2)
            wc = ((w0 if o0 else m0) * (w1 if o1 else m1)) * (w2 if o2 else m2)
            idx_buf[4 * c + (g >> 3), pl.ds(col, 16)] = idx
            wc_v[pl.ds(c * _CH + g * 16, 16)] = wc

    def pass2_group(g, l):
        cc0 = c0_v[pl.ds(g * 16, 16)]
        cc1 = c1_v[pl.ds(g * 16, 16)]
        cc2 = c2_v[pl.ds(g * 16, 16)]
        cc3 = c3_v[pl.ds(g * 16, 16)]
        f0 = jnp.zeros((16,), jnp.float32)
        f1 = jnp.zeros((16,), jnp.float32)
        for c in range(8):
            wc = wc_v[pl.ds(c * _CH + g * 16, 16)]
            coef = (wc * cc0, wc * cc1, wc * cc2, wc * cc3)
            d0 = c * _CH + g * 16 + iota
            for j in range(8):
                d2 = jnp.full((16,), j, jnp.int32)
                v = plsc.load_gather(rows_v, [d0, d2])
                if j & 1:
                    f1 = f1 + coef[j >> 1] * v
                else:
                    f0 = f0 + coef[j >> 1] * v
        out_t[2 * l, pl.ds(g * 16, 16)] = f0
        out_t[2 * l + 1, pl.ds(g * 16, 16)] = f1

    def do_level(l, dense):
        resf = resf_v[l]
        lbase = l * _T
        stride = stridei_v[l] if dense else None

        def p1(g, carry):
            pass1_group(g, l, lbase, resf, stride)
            return carry
        lax.fori_loop(0, _NG, p1, 0)

        def fire(k, carry):
            pltpu.make_async_copy(
                tbl_hbm.at[idx_buf.at[k]],
                rows_v.at[pl.ds(k * 128, 128)], sem).start()
            return carry
        lax.fori_loop(0, _IDX_ROWS, fire, 0)

        def drain(k, carry):
            pltpu.make_async_copy(
                tbl_hbm.at[idx_buf.at[k]],
                rows_v.at[pl.ds(k * 128, 128)], sem).wait()
            return carry
        lax.fori_loop(0, _IDX_ROWS, drain, 0)

        def p2(g, carry):
            pass2_group(g, l)
            return carry
        lax.fori_loop(0, _NG, p2, 0)

    def sub_chunk(s, carry):
        base = wid * _BW + s * _CH
        pltpu.sync_copy(x0_hbm.at[pl.ds(base, _CH)], x0_v)
        pltpu.sync_copy(x1_hbm.at[pl.ds(base, _CH)], x1_v)
        pltpu.sync_copy(x2_hbm.at[pl.ds(base, _CH)], x2_v)
        pltpu.sync_copy(k0_hbm.at[pl.ds(base, _CH)], c0_v)
        pltpu.sync_copy(k1_hbm.at[pl.ds(base, _CH)], c1_v)
        pltpu.sync_copy(k2_hbm.at[pl.ds(base, _CH)], c2_v)
        pltpu.sync_copy(k3_hbm.at[pl.ds(base, _CH)], c3_v)

        def dense_level(l, c):
            do_level(l, True)
            return c
        lax.fori_loop(0, _N_DENSE, dense_level, 0)

        def hash_level(l, c):
            do_level(l, False)
            return c
        lax.fori_loop(_N_DENSE, _N_LEVELS, hash_level, 0)

        pltpu.sync_copy(out_t, out_hbm.at[:, pl.ds(base, _CH)])
        return carry

    lax.fori_loop(0, _NSUB, sub_chunk, 0)


_sc_call = functools.partial(
    pl.kernel,
    mesh=plsc.VectorSubcoreMesh(core_axis_name="c", subcore_axis_name="s"),
    compiler_params=pltpu.CompilerParams(
        needs_layout_passes=False, use_tc_tiling_on_sc=False),
    out_type=jax.ShapeDtypeStruct((2 * _N_LEVELS, _B), jnp.float32),
    scratch_types=[
        pltpu.VMEM((_IDX_ROWS, 128), jnp.int32),
        pltpu.VMEM((_NIDX, 8), jnp.float32),
        pltpu.VMEM((_NIDX,), jnp.float32),
        pltpu.VMEM((_CH,), jnp.float32),
        pltpu.VMEM((_CH,), jnp.float32),
        pltpu.VMEM((_CH,), jnp.float32),
        pltpu.VMEM((_CH,), jnp.float32),
        pltpu.VMEM((_CH,), jnp.float32),
        pltpu.VMEM((_CH,), jnp.float32),
        pltpu.VMEM((_CH,), jnp.float32),
        pltpu.VMEM((2 * _N_LEVELS, _CH), jnp.float32),
        pltpu.VMEM((_N_LEVELS,), jnp.float32),
        pltpu.SMEM((_N_LEVELS,), jnp.float32),
        pltpu.SMEM((_N_LEVELS,), jnp.int32),
        pltpu.SemaphoreType.DMA,
    ],
)(_sc_body)


def kernel(in_tensor, conditioning_code, tables):
    tbl8 = jnp.transpose(tables, (1, 2, 0, 3)).reshape(_N_LEVELS * _T, _N_HASH * _F)
    resf = jnp.asarray([float(r) for r in _RES], jnp.float32)
    stridei = jnp.asarray([float(r + 1) for r in _RES], jnp.float32)
    outT = _sc_call(tbl8,
                    in_tensor[:, 0], in_tensor[:, 1], in_tensor[:, 2],
                    conditioning_code[:, 0], conditioning_code[:, 1],
                    conditioning_code[:, 2], conditioning_code[:, 3],
                    resf, stridei)
    return outT.T


# speedup vs baseline: 2.2268x; 1.0759x over previous
import functools
import numpy as np
import jax
import jax.numpy as jnp
from jax import lax
from jax.experimental import pallas as pl
from jax.experimental.pallas import tpu as pltpu
from jax.experimental.pallas import tpu_sc as plsc

_N_LEVELS = 16
_T = 2 ** 19
_BASE_RES = 16
_SCALE = 1.4472692012786865
_N_HASH = 4
_MASK = _T - 1
_P1 = -1640531535
_P2 = 805459861

_RES = [int(np.floor(_BASE_RES * _SCALE ** l)) for l in range(_N_LEVELS)]
_N_DENSE = sum(1 for r in _RES if (r + 1) ** 3 <= _T)

_B = 131072
_NC, _NS = 2, 16
_NW = _NC * _NS
_BW = _B // _NW
_CH = 512
_NSUB = _BW // _CH
_NG = _CH // 16
_NIDX = _CH * 8
_IDX_ROWS = _NIDX // 128


def _sc_body(tbl_hbm, x_hbm, code_hbm, resf_hbm, stridef_hbm, out_hbm,
             x_st, code_st, idx0, idx1, rows0, rows1, wc0, wc1,
             out_cv, lvl_stage, resf_s, stride_s, sem0, sem1):
    wid = lax.axis_index("s") * _NC + lax.axis_index("c")
    iota = lax.iota(jnp.int32, 16)
    iota3 = iota * 3
    iota4 = iota * 4

    pltpu.sync_copy(resf_hbm, lvl_stage)
    rv = lvl_stage[pl.ds(0, 16)]
    pltpu.sync_copy(stridef_hbm, lvl_stage)
    sv = lvl_stage[pl.ds(0, 16)]
    for l in range(_N_LEVELS):
        resf_s[l] = rv[l]
        stride_s[l] = sv[l]

    idx_bufs = (idx0, idx1)
    rows_bufs = (rows0, rows1)
    wc_bufs = (wc0, wc1)
    sems = (sem0, sem1)

    def pass1(l, par):
        idx_buf = idx_bufs[par]
        wc_v = wc_bufs[par]
        resf = resf_s[l]
        stridef = stride_s[l]
        stride = stridef.astype(jnp.int32)
        s2 = stride * stride
        lbase = l * _T
        is_dense = l < _N_DENSE

        def p1(g, carry):
            x0 = plsc.load_gather(x_st, [iota3 + g * 48])
            x1 = plsc.load_gather(x_st, [iota3 + (g * 48 + 1)])
            x2 = plsc.load_gather(x_st, [iota3 + (g * 48 + 2)])
            p0 = (x0 * resf).astype(jnp.int32)
            p1i = (x1 * resf).astype(jnp.int32)
            p2 = (x2 * resf).astype(jnp.int32)
            w0 = x0 * resf - p0.astype(jnp.float32)
            w1 = x1 * resf - p1i.astype(jnp.float32)
            w2 = x2 * resf - p2.astype(jnp.float32)
            m0 = 1.0 - w0
            m1 = 1.0 - w1
            m2 = 1.0 - w2
            a0 = p0
            b0 = p0 + 1
            a1 = p1i * _P1
            b1 = a1 + _P1
            a2 = p2 * _P2
            b2 = a2 + _P2
            dbase = p0 + p1i * stride + p2 * s2 + lbase
            col = (g & 7) * 16
            row0 = g >> 3
            for c in range(8):
                o0, o1, o2 = c & 1, (c >> 1) & 1, (c >> 2) & 1
                h = (b0 if o0 else a0) ^ (b1 if o1 else a1) ^ (b2 if o2 else a2)
                idx_h = (h & _MASK) + lbase
                idx_d = dbase + (o0 + stride * o1 + s2 * o2)
                idx = jnp.where(is_dense, idx_d, idx_h)
                wc = ((w0 if o0 else m0) * (w1 if o1 else m1)) * (w2 if o2 else m2)
                idx_buf[4 * c + row0, pl.ds(col, 16)] = idx
                wc_v[pl.ds(c * _CH + g * 16, 16)] = wc
            return carry
        lax.fori_loop(0, _NG, p1, 0)

    def fire(par):
        idx_buf = idx_bufs[par]
        rows_v = rows_bufs[par]
        sem = sems[par]

        def f(k, carry):
            pltpu.make_async_copy(
                tbl_hbm.at[idx_buf.at[k]],
                rows_v.at[pl.ds(k * 128, 128)], sem).start()
            return carry
        lax.fori_loop(0, _IDX_ROWS, f, 0)

    def drain(par):
        idx_buf = idx_bufs[par]
        rows_v = rows_bufs[par]
        sem = sems[par]

        def f(k, carry):
            pltpu.make_async_copy(
                tbl_hbm.at[idx_buf.at[k]],
                rows_v.at[pl.ds(k * 128, 128)], sem).wait()
            return carry
        lax.fori_loop(0, _IDX_ROWS, f, 0)

    def pass2(l, par):
        rows_v = rows_bufs[par]
        wc_v = wc_bufs[par]

        def p2(g, carry):
            cc0 = plsc.load_gather(code_st, [iota4 + g * 64])
            cc1 = plsc.load_gather(code_st, [iota4 + (g * 64 + 1)])
            cc2 = plsc.load_gather(code_st, [iota4 + (g * 64 + 2)])
            cc3 = plsc.load_gather(code_st, [iota4 + (g * 64 + 3)])
            f0 = jnp.zeros((16,), jnp.float32)
            f1 = jnp.zeros((16,), jnp.float32)
            for c in range(8):
                wc = wc_v[pl.ds(c * _CH + g * 16, 16)]
                coef = (wc * cc0, wc * cc1, wc * cc2, wc * cc3)
                d0 = c * _CH + g * 16 + iota
                for j in range(8):
                    d2 = jnp.full((16,), j, jnp.int32)
                    v = plsc.load_gather(rows_v, [d0, d2])
                    if j & 1:
                        f1 = f1 + coef[j >> 1] * v
                    else:
                        f0 = f0 + coef[j >> 1] * v
            rowv = g * 16 + iota
            plsc.store_scatter(
                out_cv, [rowv, jnp.full((16,), 2 * l, jnp.int32)], f0)
            plsc.store_scatter(
                out_cv, [rowv, jnp.full((16,), 2 * l + 1, jnp.int32)], f1)
            return carry
        lax.fori_loop(0, _NG, p2, 0)

    def sub_chunk(s, carry):
        base = wid * _BW + s * _CH
        pltpu.sync_copy(x_hbm.at[pl.ds(base * 3, 3 * _CH)], x_st)
        pltpu.sync_copy(code_hbm.at[pl.ds(base * 4, 4 * _CH)], code_st)

        pass1(0, 0)
        fire(0)

        def pair(i, c):
            l = 2 * i
            pass1(l + 1, 1)
            fire(1)
            drain(0)
            pass2(l, 0)

            @pl.when(l + 2 < _N_LEVELS)
            def _():
                pass1(l + 2, 0)
                fire(0)
            drain(1)
            pass2(l + 1, 1)
            return c
        lax.fori_loop(0, _N_LEVELS // 2, pair, 0)

        pltpu.sync_copy(out_cv, out_hbm.at[pl.ds(base, _CH), :])
        return carry

    lax.fori_loop(0, _NSUB, sub_chunk, 0)


_sc_call = functools.partial(
    pl.kernel,
    mesh=plsc.VectorSubcoreMesh(core_axis_name="c", subcore_axis_name="s"),
    compiler_params=pltpu.CompilerParams(
        needs_layout_passes=False, use_tc_tiling_on_sc=False),
    out_type=jax.ShapeDtypeStruct((_B, 2 * _N_LEVELS), jnp.float32),
    scratch_types=[
        pltpu.VMEM((3 * _CH,), jnp.float32),
        pltpu.VMEM((4 * _CH,), jnp.float32),
        pltpu.VMEM((_IDX_ROWS, 128), jnp.int32),
        pltpu.VMEM((_IDX_ROWS, 128), jnp.int32),
        pltpu.VMEM((_NIDX, 8), jnp.float32),
        pltpu.VMEM((_NIDX, 8), jnp.float32),
        pltpu.VMEM((_NIDX,), jnp.float32),
        pltpu.VMEM((_NIDX,), jnp.float32),
        pltpu.VMEM((_CH, 2 * _N_LEVELS), jnp.float32),
        pltpu.VMEM((_N_LEVELS,), jnp.float32),
        pltpu.SMEM((_N_LEVELS,), jnp.float32),
        pltpu.SMEM((_N_LEVELS,), jnp.float32),
        pltpu.SemaphoreType.DMA,
        pltpu.SemaphoreType.DMA,
    ],
)(_sc_body)


def kernel(in_tensor, conditioning_code, tables):
    tbl8 = jnp.transpose(tables, (1, 2, 0, 3)).reshape(_N_LEVELS * _T, _N_HASH * 2)
    resf = jnp.asarray([float(r) for r in _RES], jnp.float32)
    stridef = jnp.asarray([float(r + 1) for r in _RES], jnp.float32)
    return _sc_call(tbl8, in_tensor.reshape(-1), conditioning_code.reshape(-1),
                    resf, stridef)


# speedup vs baseline: 2.2566x; 1.0134x over previous
import functools
import numpy as np
import jax
import jax.numpy as jnp
from jax import lax
from jax.experimental import pallas as pl
from jax.experimental.pallas import tpu as pltpu
from jax.experimental.pallas import tpu_sc as plsc

_N_LEVELS = 16
_T = 2 ** 19
_BASE_RES = 16
_SCALE = 1.4472692012786865
_N_HASH = 4
_MASK = _T - 1
_P1 = -1640531535
_P2 = 805459861

_RES = [int(np.floor(_BASE_RES * _SCALE ** l)) for l in range(_N_LEVELS)]
_N_DENSE = sum(1 for r in _RES if (r + 1) ** 3 <= _T)

_B = 131072
_NC, _NS = 2, 16
_NW = _NC * _NS
_BW = _B // _NW
_CH = 512
_NSUB = _BW // _CH
_NG = _CH // 16
_NIDX = _CH * 8
_IDX_ROWS = _NIDX // 128


def _sc_body(tbl_hbm, x_hbm, code_hbm, resf_hbm, stridef_hbm, out_hbm,
             x_st, code_st, idx0, idx1, rows0, rows1, wc0, wc1,
             out_cv, lvl_stage, resf_s, stride_s, sem0, sem1):
    wid = lax.axis_index("s") * _NC + lax.axis_index("c")
    iota = lax.iota(jnp.int32, 16)
    iota3 = iota * 3
    iota4 = iota * 4

    pltpu.sync_copy(resf_hbm, lvl_stage)
    rv = lvl_stage[pl.ds(0, 16)]
    pltpu.sync_copy(stridef_hbm, lvl_stage)
    sv = lvl_stage[pl.ds(0, 16)]
    for l in range(_N_LEVELS):
        resf_s[l] = rv[l]
        stride_s[l] = sv[l]

    idx_bufs = (idx0, idx1)
    rows_bufs = (rows0, rows1)
    wc_bufs = (wc0, wc1)
    sems = (sem0, sem1)

    def pass1(l, par):
        idx_buf = idx_bufs[par]
        wc_v = wc_bufs[par]
        resf = resf_s[l]
        stridef = stride_s[l]
        stride = stridef.astype(jnp.int32)
        s2 = stride * stride
        lbase = l * _T
        is_dense = l < _N_DENSE

        def p1(g, carry):
            x0 = plsc.load_gather(x_st, [iota3 + g * 48])
            x1 = plsc.load_gather(x_st, [iota3 + (g * 48 + 1)])
            x2 = plsc.load_gather(x_st, [iota3 + (g * 48 + 2)])
            p0 = (x0 * resf).astype(jnp.int32)
            p1i = (x1 * resf).astype(jnp.int32)
            p2 = (x2 * resf).astype(jnp.int32)
            w0 = x0 * resf - p0.astype(jnp.float32)
            w1 = x1 * resf - p1i.astype(jnp.float32)
            w2 = x2 * resf - p2.astype(jnp.float32)
            m0 = 1.0 - w0
            m1 = 1.0 - w1
            m2 = 1.0 - w2
            a0 = p0
            b0 = p0 + 1
            a1 = p1i * _P1
            b1 = a1 + _P1
            a2 = p2 * _P2
            b2 = a2 + _P2
            dbase = p0 + p1i * stride + p2 * s2 + lbase
            col = (g & 7) * 16
            row0 = g >> 3
            for c in range(8):
                o0, o1, o2 = c & 1, (c >> 1) & 1, (c >> 2) & 1
                h = (b0 if o0 else a0) ^ (b1 if o1 else a1) ^ (b2 if o2 else a2)
                idx_h = (h & _MASK) + lbase
                idx_d = dbase + (o0 + stride * o1 + s2 * o2)
                idx = jnp.where(is_dense, idx_d, idx_h)
                wc = ((w0 if o0 else m0) * (w1 if o1 else m1)) * (w2 if o2 else m2)
                idx_buf[4 * c + row0, pl.ds(col, 16)] = idx
                wc_v[pl.ds(c * _CH + g * 16, 16)] = wc
            return carry
        lax.fori_loop(0, _NG, p1, 0)

    def fire(par):
        idx_buf = idx_bufs[par]
        rows_v = rows_bufs[par]
        sem = sems[par]

        def f(k, carry):
            pltpu.make_async_copy(
                tbl_hbm.at[idx_buf.at[k]],
                rows_v.at[pl.ds(k * 128, 128)], sem).start()
            return carry
        lax.fori_loop(0, _IDX_ROWS, f, 0)

    def drain(par):
        idx_buf = idx_bufs[par]
        rows_v = rows_bufs[par]
        sem = sems[par]

        def f(k, carry):
            pltpu.make_async_copy(
                tbl_hbm.at[idx_buf.at[k]],
                rows_v.at[pl.ds(k * 128, 128)], sem).wait()
            return carry
        lax.fori_loop(0, _IDX_ROWS, f, 0)

    def pass2(l, par):
        rows_v = rows_bufs[par]
        wc_v = wc_bufs[par]

        def p2(g, carry):
            cc0 = plsc.load_gather(code_st, [iota4 + g * 64])
            cc1 = plsc.load_gather(code_st, [iota4 + (g * 64 + 1)])
            cc2 = plsc.load_gather(code_st, [iota4 + (g * 64 + 2)])
            cc3 = plsc.load_gather(code_st, [iota4 + (g * 64 + 3)])
            f0 = jnp.zeros((16,), jnp.float32)
            f1 = jnp.zeros((16,), jnp.float32)
            for c in range(8):
                wc = wc_v[pl.ds(c * _CH + g * 16, 16)]
                coef = (wc * cc0, wc * cc1, wc * cc2, wc * cc3)
                d0 = c * _CH + g * 16 + iota
                for j in range(8):
                    d2 = jnp.full((16,), j, jnp.int32)
                    v = plsc.load_gather(rows_v, [d0, d2])
                    if j & 1:
                        f1 = f1 + coef[j >> 1] * v
                    else:
                        f0 = f0 + coef[j >> 1] * v
            rowv = g * 16 + iota
            plsc.store_scatter(
                out_cv, [rowv, jnp.full((16,), 2 * l, jnp.int32)], f0)
            plsc.store_scatter(
                out_cv, [rowv, jnp.full((16,), 2 * l + 1, jnp.int32)], f1)
            return carry
        lax.fori_loop(0, _NG, p2, 0)

    def sub_chunk(s, carry):
        base = wid * _BW + s * _CH
        pltpu.sync_copy(x_hbm.at[pl.ds(base * 3, 3 * _CH)], x_st)
        pltpu.sync_copy(code_hbm.at[pl.ds(base * 4, 4 * _CH)], code_st)

        pass1(0, 0)
        fire(0)

        def pair(i, c):
            l = 2 * i
            pass1(l + 1, 1)
            fire(1)
            drain(0)
            pass2(l, 0)

            @pl.when(l + 2 < _N_LEVELS)
            def _():
                pass1(l + 2, 0)
                fire(0)
            drain(1)
            pass2(l + 1, 1)
            return c
        lax.fori_loop(0, _N_LEVELS // 2, pair, 0)

        pltpu.sync_copy(out_cv, out_hbm.at[pl.ds(base, _CH), :])
        return carry

    lax.fori_loop(0, _NSUB, sub_chunk, 0)


_sc_call = functools.partial(
    pl.kernel,
    mesh=plsc.VectorSubcoreMesh(core_axis_name="c", subcore_axis_name="s"),
    compiler_params=pltpu.CompilerParams(
        needs_layout_passes=False, use_tc_tiling_on_sc=False),
    out_type=jax.ShapeDtypeStruct((_B, 2 * _N_LEVELS), jnp.float32),
    scratch_types=[
        pltpu.VMEM((3 * _CH,), jnp.float32),
        pltpu.VMEM((4 * _CH,), jnp.float32),
        pltpu.VMEM((_IDX_ROWS, 128), jnp.int32),
        pltpu.VMEM((_IDX_ROWS, 128), jnp.int32),
        pltpu.VMEM((_NIDX, 8), jnp.float32),
        pltpu.VMEM((_NIDX, 8), jnp.float32),
        pltpu.VMEM((_NIDX,), jnp.float32),
        pltpu.VMEM((_NIDX,), jnp.float32),
        pltpu.VMEM((_CH, 2 * _N_LEVELS), jnp.float32),
        pltpu.VMEM((_N_LEVELS,), jnp.float32),
        pltpu.SMEM((_N_LEVELS,), jnp.float32),
        pltpu.SMEM((_N_LEVELS,), jnp.float32),
        pltpu.SemaphoreType.DMA,
        pltpu.SemaphoreType.DMA,
    ],
)(_sc_body)


def kernel(in_tensor, conditioning_code, tables):
    tbl8 = jnp.concatenate(
        [tables[h].reshape(_N_LEVELS * _T, 2) for h in range(_N_HASH)], axis=1)
    resf = jnp.asarray([float(r) for r in _RES], jnp.float32)
    stridef = jnp.asarray([float(r + 1) for r in _RES], jnp.float32)
    return _sc_call(tbl8, in_tensor.reshape(-1), conditioning_code.reshape(-1),
                    resf, stridef)
